# baseline (speedup 1.0000x reference)
import functools

import jax
import jax.numpy as jnp
from jax import lax
from jax.experimental import pallas as pl
from jax.experimental.pallas import tpu as pltpu
from jax.experimental.pallas import tpu_sc as plsc

_K = 1_000_000
_D = 64
_B = 16384
_TOTAL = _B * _D
_HALF = (_K // 2) * _D
_SENT = -8


def _sc_gather(tbl_a, tbl_b, x_flat):
    info = plsc.get_sparse_core_info()
    nc, ns = info.num_cores, info.num_subcores
    nw = nc * ns
    cpw = _TOTAL // nw

    mesh = plsc.VectorSubcoreMesh(core_axis_name="c", subcore_axis_name="s")

    @functools.partial(
        pl.kernel,
        mesh=mesh,
        out_type=jax.ShapeDtypeStruct((_TOTAL,), jnp.float32),
        scratch_types=[
            pltpu.VMEM((cpw,), jnp.int32),
            pltpu.VMEM((cpw,), jnp.int32),
            pltpu.VMEM((cpw,), jnp.float32),
            pltpu.SemaphoreType.DMA,
        ],
    )
    def k(ta_hbm, tb_hbm, x_hbm, out_hbm, ia_v, ib_v, val_v, sem):
        wid = lax.axis_index("s") * nc + lax.axis_index("c")
        base = wid * cpw
        pltpu.sync_copy(x_hbm.at[pl.ds(base, cpw)], ia_v)

        lanes = lax.iota(jnp.int32, 16)

        def cbody(g, carry):
            p = g * _D
            for c0 in range(0, _D, 16):
                j = lanes + c0
                v = ia_v[pl.ds(p + c0, 16)]
                flat = (v << 6) + j
                in_a = flat < _HALF
                ia_v[pl.ds(p + c0, 16)] = jnp.where(in_a, flat, _SENT)
                ib_v[pl.ds(p + c0, 16)] = jnp.where(in_a, _SENT, flat - _HALF)
            return carry

        lax.fori_loop(0, cpw // _D, cbody, 0)

        ca = pltpu.async_copy(
            ta_hbm.at[plsc.Indices(ia_v, ignored_value=_SENT)], val_v, sem
        )
        cb = pltpu.async_copy(
            tb_hbm.at[plsc.Indices(ib_v, ignored_value=_SENT)], val_v, sem
        )
        ca.wait()
        cb.wait()

        pltpu.sync_copy(val_v, out_hbm.at[pl.ds(base, cpw)])

    return k(tbl_a, tbl_b, x_flat)


def kernel(centers, x):
    tbl_a = centers[: _K // 2].reshape(_HALF)
    tbl_b = centers[_K // 2 :].reshape(_HALF)
    x_flat = x.astype(jnp.int32).reshape(_TOTAL)
    out = _sc_gather(tbl_a, tbl_b, x_flat)
    return out.reshape(_B, _D)


# speedup vs baseline: 1.4887x; 1.4887x over previous
import functools

import jax
import jax.numpy as jnp
from jax import lax
from jax.experimental import pallas as pl
from jax.experimental.pallas import tpu as pltpu
from jax.experimental.pallas import tpu_sc as plsc

_K = 1_000_000
_D = 64
_B = 16384
_TOTAL = _B * _D


def _sc_gather(centers_flat, x_flat):
    info = plsc.get_sparse_core_info()
    nc, ns = info.num_cores, info.num_subcores
    nw = nc * ns
    cpw = _TOTAL // nw

    mesh = plsc.VectorSubcoreMesh(core_axis_name="c", subcore_axis_name="s")

    @functools.partial(
        pl.kernel,
        mesh=mesh,
        out_type=jax.ShapeDtypeStruct((_TOTAL,), jnp.float32),
        scratch_types=[
            pltpu.VMEM((cpw,), jnp.int32),
            pltpu.VMEM((cpw,), jnp.float32),
            pltpu.SemaphoreType.DMA,
        ],
    )
    def k(tbl_hbm, x_hbm, out_hbm, idx_v, val_v, sem):
        wid = lax.axis_index("s") * nc + lax.axis_index("c")
        base = wid * cpw
        pltpu.sync_copy(x_hbm.at[pl.ds(base, cpw)], idx_v)

        lanes = lax.iota(jnp.int32, 16)

        def cbody(g, carry):
            p = g * _D
            for c0 in range(0, _D, 16):
                j = lanes + c0
                v = idx_v[pl.ds(p + c0, 16)]
                idx_v[pl.ds(p + c0, 16)] = (v << 6) + j
            return carry

        lax.fori_loop(0, cpw // _D, cbody, 0)

        pltpu.async_copy(tbl_hbm.at[idx_v], val_v, sem).wait()

        pltpu.sync_copy(val_v, out_hbm.at[pl.ds(base, cpw)])

    return k(centers_flat, x_flat)


def kernel(centers, x):
    centers_flat = centers.reshape(_K * _D)
    x_flat = x.astype(jnp.int32).reshape(_TOTAL)
    out = _sc_gather(centers_flat, x_flat)
    return out.reshape(_B, _D)


# speedup vs baseline: 1.4954x; 1.0045x over previous
import functools

import jax
import jax.numpy as jnp
from jax import lax
from jax.experimental import pallas as pl
from jax.experimental.pallas import tpu as pltpu
from jax.experimental.pallas import tpu_sc as plsc

_K = 1_000_000
_D = 64
_B = 16384
_TOTAL = _B * _D


def _sc_gather(centers_flat, x_flat):
    info = plsc.get_sparse_core_info()
    nc, ns = info.num_cores, info.num_subcores
    nw = nc * ns
    cpw = _TOTAL // nw

    mesh = plsc.VectorSubcoreMesh(core_axis_name="c", subcore_axis_name="s")

    @functools.partial(
        pl.kernel,
        mesh=mesh,
        out_type=jax.ShapeDtypeStruct((_TOTAL,), jnp.float32),
        scratch_types=[
            pltpu.VMEM((cpw,), jnp.int32),
            pltpu.VMEM((cpw,), jnp.float32),
            pltpu.SemaphoreType.DMA,
        ],
    )
    def k(tbl_hbm, x_hbm, out_hbm, idx_v, val_v, sem):
        wid = lax.axis_index("s") * nc + lax.axis_index("c")
        base = wid * cpw
        pltpu.sync_copy(x_hbm.at[pl.ds(base, cpw)], idx_v)

        lanes = lax.iota(jnp.int32, 16)

        nwin = 4
        wsz = cpw // nwin

        def cbody(g, carry):
            p = g * _D
            for c0 in range(0, _D, 16):
                j = lanes + c0
                v = idx_v[pl.ds(p + c0, 16)]
                idx_v[pl.ds(p + c0, 16)] = (v << 6) + j
            return carry

        copies = []
        for w in range(nwin):
            lax.fori_loop(w * (wsz // _D), (w + 1) * (wsz // _D), cbody, 0)
            copies.append(
                pltpu.async_copy(
                    tbl_hbm.at[idx_v.at[pl.ds(w * wsz, wsz)]],
                    val_v.at[pl.ds(w * wsz, wsz)],
                    sem,
                )
            )
        for cp in copies:
            cp.wait()

        pltpu.sync_copy(val_v, out_hbm.at[pl.ds(base, cpw)])

    return k(centers_flat, x_flat)


def kernel(centers, x):
    centers_flat = centers.reshape(_K * _D)
    x_flat = x.astype(jnp.int32).reshape(_TOTAL)
    out = _sc_gather(centers_flat, x_flat)
    return out.reshape(_B, _D)
